# Initial kernel scaffold; baseline (speedup 1.0000x reference)
import jax
import jax.numpy as jnp
from jax.experimental import pallas as pl


def kernel(h, x, a, edge_index, We1, be1, We2, be2, Wa, ba, Wn1, bn1, Wn2, bn2, Wc1, bc1, Wc2, bc2, Wc3):
    raise NotImplementedError("write your pallas kernel here")


# speedup vs baseline: 2.6278x; 2.6278x over previous
import functools

import jax
import jax.numpy as jnp
from jax import lax
from jax.experimental import pallas as pl
from jax.experimental.pallas import tpu as pltpu
from jax.experimental.pallas import tpu_sc as plsc

N = 10000
E = 320000
DH = 128
DC = 3
DE = 16
TW = 144
NC = 2
NS = 16
NW = NC * NS
EW = E // NW
KB = 400
SKB = 80
NZ = N // NS
ZCH = 25
BE = 2000
BN = 2000

_f32 = jnp.float32
_bf16 = jnp.bfloat16


def _sc_gather(table, src, dst):
    mesh = plsc.VectorSubcoreMesh(core_axis_name="c", subcore_axis_name="s")

    @functools.partial(
        pl.kernel,
        out_type=(jax.ShapeDtypeStruct((E, TW), _f32),
                  jax.ShapeDtypeStruct((E, TW), _f32)),
        mesh=mesh,
        compiler_params=pltpu.CompilerParams(use_tc_tiling_on_sc=False),
        scratch_types=[
            pltpu.VMEM((KB,), jnp.int32),
            pltpu.VMEM((KB,), jnp.int32),
            pltpu.VMEM((KB, TW), _f32),
            pltpu.VMEM((KB, TW), _f32),
            pltpu.SemaphoreType.DMA,
            pltpu.SemaphoreType.DMA,
        ],
    )
    def k(table_hbm, src_hbm, dst_hbm, gs_hbm, gd_hbm,
          si_v, di_v, rs_v, rd_v, sem_s, sem_d):
        wid = lax.axis_index("s") * NC + lax.axis_index("c")
        base = wid * EW

        @pl.loop(0, EW, step=KB)
        def _(off):
            b = base + off
            pltpu.sync_copy(src_hbm.at[pl.ds(b, KB)], si_v)
            pltpu.sync_copy(dst_hbm.at[pl.ds(b, KB)], di_v)
            cps = pltpu.async_copy(table_hbm.at[si_v], rs_v, sem_s)
            cpd = pltpu.async_copy(table_hbm.at[di_v], rd_v, sem_d)
            cps.wait()
            cpd.wait()
            pltpu.sync_copy(rs_v, gs_hbm.at[pl.ds(b, KB)])
            pltpu.sync_copy(rd_v, gd_hbm.at[pl.ds(b, KB)])

    return k(table, src, dst)


def _sc_scatter(msg, dst):
    mesh = plsc.VectorSubcoreMesh(core_axis_name="c", subcore_axis_name="s")

    @functools.partial(
        pl.kernel,
        out_type=jax.ShapeDtypeStruct((NC * N, TW), _f32),
        mesh=mesh,
        compiler_params=pltpu.CompilerParams(use_tc_tiling_on_sc=False),
        scratch_types=[
            pltpu.VMEM((SKB,), jnp.int32),
            pltpu.VMEM((SKB, TW), _f32),
            pltpu.VMEM((ZCH, TW), _f32),
            pltpu.VMEM_SHARED((N, TW), _f32),
        ],
    )
    def k(msg_hbm, dst_hbm, out_hbm, di_v, rows_v, zbuf_v, acc_sh):
        c = lax.axis_index("c")
        s = lax.axis_index("s")
        wid = s * NC + c

        @pl.loop(0, ZCH)
        def _(i):
            @pl.loop(0, TW, step=16)
            def _(j):
                zbuf_v[i, pl.ds(j, 16)] = jnp.zeros((16,), _f32)

        @pl.loop(0, NZ, step=ZCH)
        def _(r):
            pltpu.sync_copy(zbuf_v, acc_sh.at[pl.ds(s * NZ + r, ZCH)])

        plsc.subcore_barrier()

        base = wid * EW

        @pl.loop(0, EW, step=SKB)
        def _(off):
            b = base + off
            pltpu.sync_copy(dst_hbm.at[pl.ds(b, SKB)], di_v)
            pltpu.sync_copy(msg_hbm.at[pl.ds(b, SKB)], rows_v)
            pltpu.sync_copy(rows_v, acc_sh.at[di_v], add=True)

        plsc.subcore_barrier()
        pltpu.sync_copy(acc_sh.at[pl.ds(s * NZ, NZ)],
                        out_hbm.at[pl.ds(c * N + s * NZ, NZ)])

    return k(msg, dst)


def _edge_compute(gs, gd, a, we1s, we1d, we1r, we1a, be1, we2, be2, wa, ba,
                  wc1s, wc1d, wc1r, wc1a, bc1, wc2, bc2, wc3):
    def body(gs_ref, gd_ref, a_ref,
             we1s_ref, we1d_ref, we1r_ref, we1a_ref, be1_ref,
             we2_ref, be2_ref, wa_ref, ba_ref,
             wc1s_ref, wc1d_ref, wc1r_ref, wc1a_ref, bc1_ref,
             wc2_ref, bc2_ref, wc3_ref, msg_ref):
        gs_b = gs_ref[...]
        gd_b = gd_ref[...]
        hs = gs_b[:, :DH].astype(_bf16)
        hd = gd_b[:, :DH].astype(_bf16)
        xdiff = gs_b[:, DH:DH + DC] - gd_b[:, DH:DH + DC]
        radial = jnp.sqrt(jnp.sum(xdiff * xdiff, axis=1, keepdims=True))
        xdn = xdiff / (radial + 1.0)
        ab = a_ref[...].astype(_bf16)

        def pre1(ws_ref, wd_ref, wr_ref, wa2_ref, b_ref):
            p = jnp.dot(hs, ws_ref[...], preferred_element_type=_f32)
            p = p + jnp.dot(hd, wd_ref[...], preferred_element_type=_f32)
            p = p + jnp.dot(ab, wa2_ref[...], preferred_element_type=_f32)
            return p + radial * wr_ref[...] + b_ref[...]

        mh = jax.nn.silu(pre1(we1s_ref, we1d_ref, we1r_ref, we1a_ref, be1_ref))
        mh = jax.nn.silu(jnp.dot(mh.astype(_bf16), we2_ref[...],
                                 preferred_element_type=_f32) + be2_ref[...])
        att = jax.nn.sigmoid(jnp.dot(mh.astype(_bf16), wa_ref[...],
                                     preferred_element_type=_f32) + ba_ref[...])
        msg_h = att * mh
        ch = jax.nn.silu(pre1(wc1s_ref, wc1d_ref, wc1r_ref, wc1a_ref, bc1_ref))
        ch = jax.nn.silu(jnp.dot(ch.astype(_bf16), wc2_ref[...],
                                 preferred_element_type=_f32) + bc2_ref[...])
        coef = jnp.dot(ch.astype(_bf16), wc3_ref[...],
                       preferred_element_type=_f32)
        msg_x = coef * xdn
        msg_ref[...] = jnp.concatenate(
            [msg_h, msg_x, jnp.zeros((BE, TW - DH - DC), _f32)], axis=1)

    full = lambda arr: pl.BlockSpec(arr.shape, lambda i: (0,) * arr.ndim)
    return pl.pallas_call(
        body,
        grid=(E // BE,),
        in_specs=[
            pl.BlockSpec((BE, TW), lambda i: (i, 0)),
            pl.BlockSpec((BE, TW), lambda i: (i, 0)),
            pl.BlockSpec((BE, DE), lambda i: (i, 0)),
            full(we1s), full(we1d), full(we1r), full(we1a), full(be1),
            full(we2), full(be2), full(wa), full(ba),
            full(wc1s), full(wc1d), full(wc1r), full(wc1a), full(bc1),
            full(wc2), full(bc2), full(wc3),
        ],
        out_specs=pl.BlockSpec((BE, TW), lambda i: (i, 0)),
        out_shape=jax.ShapeDtypeStruct((E, TW), _f32),
    )(gs, gd, a, we1s, we1d, we1r, we1a, be1, we2, be2, wa, ba,
      wc1s, wc1d, wc1r, wc1a, bc1, wc2, bc2, wc3)


def _node_compute(h, x, p0, p1, wn1h, wn1n, bn1, wn2, bn2):
    def body(h_ref, x_ref, p0_ref, p1_ref,
             wn1h_ref, wn1n_ref, bn1_ref, wn2_ref, bn2_ref,
             ho_ref, xo_ref):
        hn = p0_ref[:, :DH] + p1_ref[:, :DH]
        xn = p0_ref[:, DH:DH + DC] + p1_ref[:, DH:DH + DC]
        h_b = h_ref[...]
        pre = (jnp.dot(h_b.astype(_bf16), wn1h_ref[...],
                       preferred_element_type=_f32)
               + jnp.dot(hn.astype(_bf16), wn1n_ref[...],
                         preferred_element_type=_f32)
               + bn1_ref[...])
        nh = jax.nn.silu(pre)
        nh = jnp.dot(nh.astype(_bf16), wn2_ref[...],
                     preferred_element_type=_f32) + bn2_ref[...]
        ho_ref[...] = h_b + nh
        xo_ref[...] = x_ref[...] + xn

    full = lambda arr: pl.BlockSpec(arr.shape, lambda i: (0,) * arr.ndim)
    return pl.pallas_call(
        body,
        grid=(N // BN,),
        in_specs=[
            pl.BlockSpec((BN, DH), lambda i: (i, 0)),
            pl.BlockSpec((BN, DC), lambda i: (i, 0)),
            pl.BlockSpec((BN, TW), lambda i: (i, 0)),
            pl.BlockSpec((BN, TW), lambda i: (i, 0)),
            full(wn1h), full(wn1n), full(bn1), full(wn2), full(bn2),
        ],
        out_specs=[
            pl.BlockSpec((BN, DH), lambda i: (i, 0)),
            pl.BlockSpec((BN, DC), lambda i: (i, 0)),
        ],
        out_shape=[
            jax.ShapeDtypeStruct((N, DH), _f32),
            jax.ShapeDtypeStruct((N, DC), _f32),
        ],
    )(h, x, p0, p1, wn1h, wn1n, bn1, wn2, bn2)


def kernel(h, x, a, edge_index, We1, be1, We2, be2, Wa, ba, Wn1, bn1, Wn2,
           bn2, Wc1, bc1, Wc2, bc2, Wc3):
    src = edge_index[0]
    dst = edge_index[1]
    table = jnp.concatenate(
        [h, x, jnp.zeros((N, TW - DH - DC), _f32)], axis=1)

    gs, gd = _sc_gather(table, src, dst)

    bf = lambda w: w.astype(_bf16)
    row = lambda b: b.reshape(1, -1)
    msg = _edge_compute(
        gs, gd, a,
        bf(We1[:DH]), bf(We1[DH:2 * DH]), We1[2 * DH:2 * DH + 1],
        bf(We1[2 * DH + 1:]), row(be1),
        bf(We2), row(be2), bf(Wa), row(ba),
        bf(Wc1[:DH]), bf(Wc1[DH:2 * DH]), Wc1[2 * DH:2 * DH + 1],
        bf(Wc1[2 * DH + 1:]), row(bc1),
        bf(Wc2), row(bc2), bf(Wc3))

    parts = _sc_scatter(msg, dst)
    p0 = parts[:N]
    p1 = parts[N:]

    h_out, x_out = _node_compute(h, x, p0, p1, bf(Wn1[:DH]), bf(Wn1[DH:]),
                                 row(bn1), bf(Wn2), row(bn2))
    return (h_out, x_out)
